# baseline (device time: 263630 ns/iter reference)
import jax
import jax.numpy as jnp
from jax import lax
from jax.experimental import pallas as pl
from jax.experimental.pallas import tpu as pltpu

N_DEV = 4
HQ = 8
DH = 128
SQ = 2048
G = 4
GR = SQ // G
DM = 1024
DC = 1024
SCALE = 0.08838834764831843


def _group_rows(t):
    n = t.shape[0]
    rest = t.shape[1:]
    return (
        t.reshape(8, 4, 64, *rest).swapaxes(0, 1).reshape(n, *rest)
    )


def kernel(x, Wq, K_ext, V_ext, Wo):
    xg = _group_rows(x[0]).astype(jnp.bfloat16)
    wq_b = Wq.astype(jnp.bfloat16)
    wo_b = Wo.astype(jnp.bfloat16)

    def prep_kv(t):
        u = t[0].transpose(1, 0, 2)
        u = u.reshape(32, 8, 4, 64, DH).swapaxes(1, 2)
        return u.reshape(32, G, GR, DH).astype(jnp.bfloat16)

    kg = prep_kv(K_ext)
    vg = prep_kv(V_ext)

    def body(xg_ref, wq_ref, wo_ref, kg_ref, vg_ref, out_ref,
             comm_ref, qb_ref, ctx_ref, kh_ref, vh_ref,
             ssems, rsems, ksem, vsem):
        i = lax.axis_index("i")
        left = (i - 1) % N_DEV
        right = (i + 1) % N_DEV

        barrier_sem = pltpu.get_barrier_semaphore()
        for nbr in (left, right):
            pl.semaphore_signal(
                barrier_sem, inc=1,
                device_id=(nbr,), device_id_type=pl.DeviceIdType.MESH,
            )
        pl.semaphore_wait(barrier_sem, 2)

        comm_ref[0, 0] = wq_ref[:]
        comm_ref[0, 1] = wo_ref[:]

        rdmas = [
            pltpu.make_async_remote_copy(
                src_ref=comm_ref.at[t],
                dst_ref=comm_ref.at[t + 1],
                send_sem=ssems.at[t],
                recv_sem=rsems.at[t],
                device_id=(right,),
                device_id_type=pl.DeviceIdType.MESH,
            )
            for t in range(N_DEV - 1)
        ]

        for t in range(N_DEV):
            if t > 0:
                rdmas[t - 1].wait_recv()
            if t < N_DEV - 1:
                rdmas[t].start()

            j = (i - t) % N_DEV
            wq_c = comm_ref[t, 0]
            wo_c = comm_ref[t, 1]

            for r in range(G):
                qf = jnp.dot(
                    xg_ref[r * GR:(r + 1) * GR, :], wq_c,
                    preferred_element_type=jnp.float32,
                )
                qb_ref[r * GR:(r + 1) * GR, :] = (qf * SCALE).astype(
                    jnp.bfloat16)

            for hl in range(HQ):
                g = j * HQ + hl
                ck = pltpu.make_async_copy(kg_ref.at[g], kh_ref, ksem)
                cv = pltpu.make_async_copy(vg_ref.at[g], vh_ref, vsem)
                ck.start()
                cv.start()
                ck.wait()
                cv.wait()
                for r in range(G):
                    q = qb_ref[r * GR:(r + 1) * GR,
                               hl * DH:(hl + 1) * DH]
                    s = lax.dot_general(
                        q, kh_ref[r],
                        (((1,), (1,)), ((), ())),
                        preferred_element_type=jnp.float32,
                    )
                    m = jnp.max(s, axis=1, keepdims=True)
                    e = jnp.exp(s - m)
                    den = jnp.sum(e, axis=1, keepdims=True)
                    p = (e / den).astype(jnp.bfloat16)
                    c = jnp.dot(p, vh_ref[r],
                                preferred_element_type=jnp.float32)
                    ctx_ref[r * GR:(r + 1) * GR,
                            hl * DH:(hl + 1) * DH] = c.astype(jnp.bfloat16)

            for r in range(G):
                part = jnp.dot(
                    ctx_ref[r * GR:(r + 1) * GR, :], wo_c,
                    preferred_element_type=jnp.float32,
                )
                if t == 0:
                    out_ref[r] = part
                else:
                    out_ref[r] = out_ref[r] + part

        for t in range(N_DEV - 1):
            rdmas[t].wait_send()

    out_g = pl.pallas_call(
        body,
        out_shape=jax.ShapeDtypeStruct((G, GR, DM), jnp.float32),
        in_specs=[
            pl.BlockSpec(memory_space=pltpu.MemorySpace.VMEM),
            pl.BlockSpec(memory_space=pltpu.MemorySpace.VMEM),
            pl.BlockSpec(memory_space=pltpu.MemorySpace.VMEM),
            pl.BlockSpec(memory_space=pltpu.MemorySpace.HBM),
            pl.BlockSpec(memory_space=pltpu.MemorySpace.HBM),
        ],
        out_specs=pl.BlockSpec(memory_space=pltpu.MemorySpace.VMEM),
        scratch_shapes=[
            pltpu.VMEM((N_DEV, 2, DM, DC), jnp.bfloat16),
            pltpu.VMEM((SQ, DC), jnp.bfloat16),
            pltpu.VMEM((SQ, DC), jnp.bfloat16),
            pltpu.VMEM((G, GR, DH), jnp.bfloat16),
            pltpu.VMEM((G, GR, DH), jnp.bfloat16),
            pltpu.SemaphoreType.DMA((N_DEV - 1,)),
            pltpu.SemaphoreType.DMA((N_DEV - 1,)),
            pltpu.SemaphoreType.DMA,
            pltpu.SemaphoreType.DMA,
        ],
        compiler_params=pltpu.CompilerParams(
            collective_id=0,
            vmem_limit_bytes=100 * 1024 * 1024,
        ),
    )(xg, wq_b, wo_b, kg, vg)

    out = out_g.reshape(4, 8, 64, DM).swapaxes(0, 1).reshape(1, SQ, DM)
    return out


# device time: 259831 ns/iter; 1.0146x vs baseline; 1.0146x over previous
import jax
import jax.numpy as jnp
from jax import lax
from jax.experimental import pallas as pl
from jax.experimental.pallas import tpu as pltpu

N_DEV = 4
HQ = 8
DH = 128
SQ = 2048
G = 4
GR = SQ // G
DM = 1024
DC = 1024
SCALE = 0.08838834764831843


def _group_rows(t):
    n = t.shape[0]
    rest = t.shape[1:]
    return (
        t.reshape(8, 4, 64, *rest).swapaxes(0, 1).reshape(n, *rest)
    )


def kernel(x, Wq, K_ext, V_ext, Wo):
    xg = _group_rows(x[0]).astype(jnp.bfloat16)
    wq_b = Wq.astype(jnp.bfloat16)
    wo_b = Wo.astype(jnp.bfloat16)

    def prep_v(t):
        u = t[0].transpose(1, 0, 2)
        u = u.reshape(32, 8, 4, 64, DH).swapaxes(1, 2)
        return u.reshape(32, G, GR, DH).astype(jnp.bfloat16)

    def prep_kT(t):
        u = t[0].transpose(1, 2, 0)
        u = u.reshape(32, DH, 8, 4, 64).transpose(0, 3, 1, 2, 4)
        return u.reshape(32, G, DH, GR).astype(jnp.bfloat16)

    kg = prep_kT(K_ext)
    vg = prep_v(V_ext)

    def body(xg_ref, wq_ref, wo_ref, kg_ref, vg_ref, out_ref,
             comm_ref, qb_ref, ctx_ref, kh_ref, vh_ref,
             ssems, rsems, ksem, vsem):
        i = lax.axis_index("i")
        left = (i - 1) % N_DEV
        right = (i + 1) % N_DEV

        barrier_sem = pltpu.get_barrier_semaphore()
        for nbr in (left, right):
            pl.semaphore_signal(
                barrier_sem, inc=1,
                device_id=(nbr,), device_id_type=pl.DeviceIdType.MESH,
            )
        pl.semaphore_wait(barrier_sem, 2)

        comm_ref[0, 0] = wq_ref[:]
        comm_ref[0, 1] = wo_ref[:]

        rdmas = [
            pltpu.make_async_remote_copy(
                src_ref=comm_ref.at[t],
                dst_ref=comm_ref.at[t + 1],
                send_sem=ssems.at[t],
                recv_sem=rsems.at[t],
                device_id=(right,),
                device_id_type=pl.DeviceIdType.MESH,
            )
            for t in range(N_DEV - 1)
        ]

        def kv_copies(chunk, hl, buf):
            g = chunk * HQ + hl
            return (
                pltpu.make_async_copy(kg_ref.at[g], kh_ref.at[buf],
                                      ksem.at[buf]),
                pltpu.make_async_copy(vg_ref.at[g], vh_ref.at[buf],
                                      vsem.at[buf]),
            )

        ck0, cv0 = kv_copies(i, 0, 0)
        ck0.start()
        cv0.start()

        for t in range(N_DEV):
            if t > 0:
                rdmas[t - 1].wait_recv()
            if t < N_DEV - 1:
                rdmas[t].start()

            j = (i - t) % N_DEV
            wq_c = comm_ref[t, 0]
            wo_c = comm_ref[t, 1]

            for r in range(G):
                qf = jnp.dot(
                    xg_ref[r * GR:(r + 1) * GR, :], wq_c,
                    preferred_element_type=jnp.float32,
                )
                qb_ref[r * GR:(r + 1) * GR, :] = (qf * SCALE).astype(
                    jnp.bfloat16)

            for hl in range(HQ):
                buf = hl % 2
                ck, cv = kv_copies(j, hl, buf)
                ck.wait()
                cv.wait()
                if hl < HQ - 1:
                    nk, nv = kv_copies(j, hl + 1, 1 - buf)
                    nk.start()
                    nv.start()
                elif t < N_DEV - 1:
                    nk, nv = kv_copies((i - t - 1) % N_DEV, 0, 1 - buf)
                    nk.start()
                    nv.start()
                for r in range(G):
                    q = qb_ref[r * GR:(r + 1) * GR,
                               hl * DH:(hl + 1) * DH]
                    s = jnp.dot(q, kh_ref[buf, r],
                                preferred_element_type=jnp.float32)
                    m = jnp.max(s, axis=1, keepdims=True)
                    e = jnp.exp(s - m)
                    den = jnp.sum(e, axis=1, keepdims=True)
                    p = (e / den).astype(jnp.bfloat16)
                    c = jnp.dot(p, vh_ref[buf, r],
                                preferred_element_type=jnp.float32)
                    ctx_ref[r * GR:(r + 1) * GR,
                            hl * DH:(hl + 1) * DH] = c.astype(jnp.bfloat16)

            for r in range(G):
                part = jnp.dot(
                    ctx_ref[r * GR:(r + 1) * GR, :], wo_c,
                    preferred_element_type=jnp.float32,
                )
                if t == 0:
                    out_ref[r] = part
                else:
                    out_ref[r] = out_ref[r] + part

        for t in range(N_DEV - 1):
            rdmas[t].wait_send()

    out_g = pl.pallas_call(
        body,
        out_shape=jax.ShapeDtypeStruct((G, GR, DM), jnp.float32),
        in_specs=[
            pl.BlockSpec(memory_space=pltpu.MemorySpace.VMEM),
            pl.BlockSpec(memory_space=pltpu.MemorySpace.VMEM),
            pl.BlockSpec(memory_space=pltpu.MemorySpace.VMEM),
            pl.BlockSpec(memory_space=pltpu.MemorySpace.HBM),
            pl.BlockSpec(memory_space=pltpu.MemorySpace.HBM),
        ],
        out_specs=pl.BlockSpec(memory_space=pltpu.MemorySpace.VMEM),
        scratch_shapes=[
            pltpu.VMEM((N_DEV, 2, DM, DC), jnp.bfloat16),
            pltpu.VMEM((SQ, DC), jnp.bfloat16),
            pltpu.VMEM((SQ, DC), jnp.bfloat16),
            pltpu.VMEM((2, G, DH, GR), jnp.bfloat16),
            pltpu.VMEM((2, G, GR, DH), jnp.bfloat16),
            pltpu.SemaphoreType.DMA((N_DEV - 1,)),
            pltpu.SemaphoreType.DMA((N_DEV - 1,)),
            pltpu.SemaphoreType.DMA((2,)),
            pltpu.SemaphoreType.DMA((2,)),
        ],
        compiler_params=pltpu.CompilerParams(
            collective_id=0,
            vmem_limit_bytes=100 * 1024 * 1024,
        ),
    )(xg, wq_b, wo_b, kg, vg)

    out = out_g.reshape(4, 8, 64, DM).swapaxes(0, 1).reshape(1, SQ, DM)
    return out


# device time: 183720 ns/iter; 1.4350x vs baseline; 1.4143x over previous
import jax
import jax.numpy as jnp
from jax import lax
from jax.experimental import pallas as pl
from jax.experimental.pallas import tpu as pltpu

N_DEV = 4
HQ = 8
DH = 128
SQ = 2048
G = 4
GR = SQ // G
P = GR // 64
DM = 1024
DC = 1024
SCALE = 0.08838834764831843


def _group_rows(t):
    n = t.shape[0]
    rest = t.shape[1:]
    return t.reshape(P, G, 64, *rest).swapaxes(0, 1).reshape(n, *rest)


def kernel(x, Wq, K_ext, V_ext, Wo):
    xg = _group_rows(x[0]).astype(jnp.bfloat16)
    wq_b = Wq.astype(jnp.bfloat16)
    wo_b = Wo.astype(jnp.bfloat16)
    kr = K_ext.reshape(P, G, 64, 32, DH)
    vr = V_ext.reshape(P, G, 64, 32, DH)

    def body(xg_ref, wq_ref, wo_ref, kr_ref, vr_ref, out_ref,
             comm_ref, qb_ref, ctx_ref, kraw_ref, vraw_ref,
             khg_ref, vhg_ref, sqs, rqs, sos, ros, ksem, vsem):
        i = lax.axis_index("i")
        left = (i - 1) % N_DEV
        right = (i + 1) % N_DEV

        barrier_sem = pltpu.get_barrier_semaphore()
        for nbr in (left, right):
            pl.semaphore_signal(
                barrier_sem, inc=1,
                device_id=(nbr,), device_id_type=pl.DeviceIdType.MESH,
            )
        pl.semaphore_wait(barrier_sem, 2)

        comm_ref[0, 0] = wq_ref[:]
        comm_ref[0, 1] = wo_ref[:]

        def ring_rdma(t, part, ssem, rsem):
            return pltpu.make_async_remote_copy(
                src_ref=comm_ref.at[t, part],
                dst_ref=comm_ref.at[t + 1, part],
                send_sem=ssem.at[t],
                recv_sem=rsem.at[t],
                device_id=(right,),
                device_id_type=pl.DeviceIdType.MESH,
            )

        wq_rdmas = [ring_rdma(t, 0, sqs, rqs) for t in range(N_DEV - 1)]
        wo_rdmas = [ring_rdma(t, 1, sos, ros) for t in range(N_DEV - 1)]

        def kv_copies(chunk, hl, buf):
            g = chunk * HQ + hl
            return (
                pltpu.make_async_copy(
                    kr_ref.at[:, :, :, g, :], kraw_ref.at[buf],
                    ksem.at[buf]),
                pltpu.make_async_copy(
                    vr_ref.at[:, :, :, g, :], vraw_ref.at[buf],
                    vsem.at[buf]),
            )

        pending = kv_copies(i, 0, 0)
        for c in pending:
            c.start()

        for t in range(N_DEV):
            if t > 0:
                wq_rdmas[t - 1].wait_recv()
            if t < N_DEV - 1:
                wq_rdmas[t].start()

            j = (i - t) % N_DEV
            wq_c = comm_ref[t, 0]

            qf = jnp.dot(xg_ref[:], wq_c,
                         preferred_element_type=jnp.float32)
            qb_ref[:] = (qf * SCALE).astype(jnp.bfloat16)

            for hl in range(HQ):
                buf = hl % 2
                for c in pending:
                    c.wait()
                khg_ref[:] = kraw_ref[buf].swapaxes(0, 1).reshape(
                    G, GR, DH).astype(jnp.bfloat16)
                vhg_ref[:] = vraw_ref[buf].swapaxes(0, 1).reshape(
                    G, GR, DH).astype(jnp.bfloat16)
                if hl < HQ - 1:
                    pending = kv_copies(j, hl + 1, 1 - buf)
                elif t < N_DEV - 1:
                    pending = kv_copies((i - t - 1) % N_DEV, 0, 1 - buf)
                else:
                    pending = []
                for c in pending:
                    c.start()
                q = qb_ref[:, hl * DH:(hl + 1) * DH].reshape(G, GR, DH)
                s = lax.dot_general(
                    q, khg_ref[:],
                    (((2,), (2,)), ((0,), (0,))),
                    preferred_element_type=jnp.float32,
                )
                e = jnp.exp(s)
                den = jnp.sum(e, axis=2, keepdims=True)
                c_un = lax.dot_general(
                    e.astype(jnp.bfloat16), vhg_ref[:],
                    (((2,), (1,)), ((0,), (0,))),
                    preferred_element_type=jnp.float32,
                )
                c_n = c_un * (1.0 / den)
                ctx_ref[:, hl * DH:(hl + 1) * DH] = c_n.astype(
                    jnp.bfloat16).reshape(SQ, DH)

            if t > 0:
                wo_rdmas[t - 1].wait_recv()
            if t < N_DEV - 1:
                wo_rdmas[t].start()
            wo_c = comm_ref[t, 1]

            part = jnp.dot(ctx_ref[:], wo_c,
                           preferred_element_type=jnp.float32)
            if t == 0:
                out_ref[:] = part.reshape(G, GR, DM)
            else:
                out_ref[:] = out_ref[:] + part.reshape(G, GR, DM)

        for t in range(N_DEV - 1):
            wq_rdmas[t].wait_send()
            wo_rdmas[t].wait_send()

    out_g = pl.pallas_call(
        body,
        out_shape=jax.ShapeDtypeStruct((G, GR, DM), jnp.float32),
        in_specs=[
            pl.BlockSpec(memory_space=pltpu.MemorySpace.VMEM),
            pl.BlockSpec(memory_space=pltpu.MemorySpace.VMEM),
            pl.BlockSpec(memory_space=pltpu.MemorySpace.VMEM),
            pl.BlockSpec(memory_space=pltpu.MemorySpace.HBM),
            pl.BlockSpec(memory_space=pltpu.MemorySpace.HBM),
        ],
        out_specs=pl.BlockSpec(memory_space=pltpu.MemorySpace.VMEM),
        scratch_shapes=[
            pltpu.VMEM((N_DEV, 2, DM, DC), jnp.bfloat16),
            pltpu.VMEM((SQ, DC), jnp.bfloat16),
            pltpu.VMEM((SQ, DC), jnp.bfloat16),
            pltpu.VMEM((2, P, G, 64, DH), jnp.float32),
            pltpu.VMEM((2, P, G, 64, DH), jnp.float32),
            pltpu.VMEM((G, GR, DH), jnp.bfloat16),
            pltpu.VMEM((G, GR, DH), jnp.bfloat16),
            pltpu.SemaphoreType.DMA((N_DEV - 1,)),
            pltpu.SemaphoreType.DMA((N_DEV - 1,)),
            pltpu.SemaphoreType.DMA((N_DEV - 1,)),
            pltpu.SemaphoreType.DMA((N_DEV - 1,)),
            pltpu.SemaphoreType.DMA((2,)),
            pltpu.SemaphoreType.DMA((2,)),
        ],
        compiler_params=pltpu.CompilerParams(
            collective_id=0,
            vmem_limit_bytes=100 * 1024 * 1024,
        ),
    )(xg, wq_b, wo_b, kr, vr)

    out = out_g.reshape(G, P, 64, DM).swapaxes(0, 1).reshape(1, SQ, DM)
    return out


# device time: 179250 ns/iter; 1.4707x vs baseline; 1.0249x over previous
import jax
import jax.numpy as jnp
from jax import lax
from jax.experimental import pallas as pl
from jax.experimental.pallas import tpu as pltpu

N_DEV = 4
HQ = 8
DH = 128
SQ = 2048
G = 4
GR = SQ // G
P = GR // 64
DM = 1024
DC = 1024
SCALE = 0.08838834764831843


def _group_rows(t):
    n = t.shape[0]
    rest = t.shape[1:]
    return t.reshape(P, G, 64, *rest).swapaxes(0, 1).reshape(n, *rest)


def kernel(x, Wq, K_ext, V_ext, Wo):
    xg = _group_rows(x[0]).astype(jnp.bfloat16)
    wq_b = Wq.astype(jnp.bfloat16)
    wo_b = Wo.astype(jnp.bfloat16)
    kr = K_ext.reshape(P, G, 64, 32, DH)
    vr = V_ext.reshape(P, G, 64, 32, DH)

    def body(xg_ref, wq_ref, wo_ref, kr_ref, vr_ref, out_ref,
             comm_ref, qb_ref, ctx_ref, kraw_ref, vraw_ref,
             khg_ref, vhg_ref, sqs, rqs, sos, ros, ksem, vsem):
        i = lax.axis_index("i")
        left = (i - 1) % N_DEV
        right = (i + 1) % N_DEV

        barrier_sem = pltpu.get_barrier_semaphore()
        for nbr in (left, right):
            pl.semaphore_signal(
                barrier_sem, inc=1,
                device_id=(nbr,), device_id_type=pl.DeviceIdType.MESH,
            )
        pl.semaphore_wait(barrier_sem, 2)

        comm_ref[0, 0] = wq_ref[:]
        comm_ref[0, 1] = wo_ref[:]

        def ring_rdma(t, part, ssem, rsem):
            return pltpu.make_async_remote_copy(
                src_ref=comm_ref.at[t, part],
                dst_ref=comm_ref.at[t + 1, part],
                send_sem=ssem.at[t],
                recv_sem=rsem.at[t],
                device_id=(right,),
                device_id_type=pl.DeviceIdType.MESH,
            )

        wq_rdmas = [ring_rdma(t, 0, sqs, rqs) for t in range(N_DEV - 1)]
        wo_rdmas = [ring_rdma(t, 1, sos, ros) for t in range(N_DEV - 1)]

        def kv_copies(chunk, hl, buf):
            g = chunk * HQ + hl
            return (
                pltpu.make_async_copy(
                    kr_ref.at[:, :, :, g, :], kraw_ref.at[buf],
                    ksem.at[buf]),
                pltpu.make_async_copy(
                    vr_ref.at[:, :, :, g, :], vraw_ref.at[buf],
                    vsem.at[buf]),
            )

        pending = kv_copies(i, 0, 0)
        for c in pending:
            c.start()

        for t in range(N_DEV):
            if t > 0:
                wq_rdmas[t - 1].wait_recv()
            if t < N_DEV - 1:
                wq_rdmas[t].start()
            if t == 0:
                wo_rdmas[0].start()

            j = (i - t) % N_DEV
            wq_c = comm_ref[t, 0]

            qf = jnp.dot(xg_ref[:], wq_c,
                         preferred_element_type=jnp.float32)
            qb_ref[:] = (qf * SCALE).astype(jnp.bfloat16)

            for hl in range(HQ):
                buf = hl % 2
                if hl == HQ // 2 and t > 0:
                    wo_rdmas[t - 1].wait_recv()
                    if t < N_DEV - 1:
                        wo_rdmas[t].start()
                for c in pending:
                    c.wait()
                khg_ref[:] = kraw_ref[buf].swapaxes(0, 1).reshape(
                    G, GR, DH).astype(jnp.bfloat16)
                vhg_ref[:] = vraw_ref[buf].swapaxes(0, 1).reshape(
                    G, GR, DH).astype(jnp.bfloat16)
                if hl < HQ - 1:
                    pending = kv_copies(j, hl + 1, 1 - buf)
                elif t < N_DEV - 1:
                    pending = kv_copies((i - t - 1) % N_DEV, 0, 1 - buf)
                else:
                    pending = []
                for c in pending:
                    c.start()
                q = qb_ref[:, hl * DH:(hl + 1) * DH].reshape(G, GR, DH)
                s = lax.dot_general(
                    q, khg_ref[:],
                    (((2,), (2,)), ((0,), (0,))),
                    preferred_element_type=jnp.float32,
                )
                e = jnp.exp(s)
                den = jnp.sum(e, axis=2, keepdims=True)
                c_un = lax.dot_general(
                    e.astype(jnp.bfloat16), vhg_ref[:],
                    (((2,), (1,)), ((0,), (0,))),
                    preferred_element_type=jnp.float32,
                )
                c_n = c_un * (1.0 / den)
                ctx_ref[:, hl * DH:(hl + 1) * DH] = c_n.astype(
                    jnp.bfloat16).reshape(SQ, DH)

            wo_c = comm_ref[t, 1]

            part = jnp.dot(ctx_ref[:], wo_c,
                           preferred_element_type=jnp.float32)
            if t == 0:
                out_ref[:] = part.reshape(G, GR, DM)
            else:
                out_ref[:] = out_ref[:] + part.reshape(G, GR, DM)

        for t in range(N_DEV - 1):
            wq_rdmas[t].wait_send()
            wo_rdmas[t].wait_send()

    out_g = pl.pallas_call(
        body,
        out_shape=jax.ShapeDtypeStruct((G, GR, DM), jnp.float32),
        in_specs=[
            pl.BlockSpec(memory_space=pltpu.MemorySpace.VMEM),
            pl.BlockSpec(memory_space=pltpu.MemorySpace.VMEM),
            pl.BlockSpec(memory_space=pltpu.MemorySpace.VMEM),
            pl.BlockSpec(memory_space=pltpu.MemorySpace.HBM),
            pl.BlockSpec(memory_space=pltpu.MemorySpace.HBM),
        ],
        out_specs=pl.BlockSpec(memory_space=pltpu.MemorySpace.VMEM),
        scratch_shapes=[
            pltpu.VMEM((N_DEV, 2, DM, DC), jnp.bfloat16),
            pltpu.VMEM((SQ, DC), jnp.bfloat16),
            pltpu.VMEM((SQ, DC), jnp.bfloat16),
            pltpu.VMEM((2, P, G, 64, DH), jnp.float32),
            pltpu.VMEM((2, P, G, 64, DH), jnp.float32),
            pltpu.VMEM((G, GR, DH), jnp.bfloat16),
            pltpu.VMEM((G, GR, DH), jnp.bfloat16),
            pltpu.SemaphoreType.DMA((N_DEV - 1,)),
            pltpu.SemaphoreType.DMA((N_DEV - 1,)),
            pltpu.SemaphoreType.DMA((N_DEV - 1,)),
            pltpu.SemaphoreType.DMA((N_DEV - 1,)),
            pltpu.SemaphoreType.DMA((2,)),
            pltpu.SemaphoreType.DMA((2,)),
        ],
        compiler_params=pltpu.CompilerParams(
            collective_id=0,
            vmem_limit_bytes=100 * 1024 * 1024,
        ),
    )(xg, wq_b, wo_b, kr, vr)

    out = out_g.reshape(G, P, 64, DM).swapaxes(0, 1).reshape(1, SQ, DM)
    return out


# device time: 175545 ns/iter; 1.5018x vs baseline; 1.0211x over previous
import jax
import jax.numpy as jnp
from jax import lax
from jax.experimental import pallas as pl
from jax.experimental.pallas import tpu as pltpu

N_DEV = 4
HQ = 8
DH = 128
SQ = 2048
G = 4
GR = SQ // G
P = GR // 64
DM = 1024
DC = 1024
SCALE = 0.08838834764831843


def _group_rows(t):
    n = t.shape[0]
    rest = t.shape[1:]
    return t.reshape(P, G, 64, *rest).swapaxes(0, 1).reshape(n, *rest)


def kernel(x, Wq, K_ext, V_ext, Wo):
    xg = _group_rows(x[0]).astype(jnp.bfloat16)
    wq_b = Wq.astype(jnp.bfloat16)
    wo_b = Wo.astype(jnp.bfloat16)
    kr = K_ext.reshape(P, G, 64, 32, DH)
    vr = V_ext.reshape(P, G, 64, 32, DH)

    def body(xg_ref, wq_ref, wo_ref, kr_ref, vr_ref, out_ref,
             comm_ref, qb_ref, ctx_ref, kraw_ref, vraw_ref,
             khg_ref, vhg_ref, sqs, rqs, sos, ros, ksem, vsem):
        i = lax.axis_index("i")
        left = (i - 1) % N_DEV
        right = (i + 1) % N_DEV

        barrier_sem = pltpu.get_barrier_semaphore()
        for nbr in (left, right):
            pl.semaphore_signal(
                barrier_sem, inc=1,
                device_id=(nbr,), device_id_type=pl.DeviceIdType.MESH,
            )
        pl.semaphore_wait(barrier_sem, 2)

        comm_ref[0, 0] = wq_ref[:]
        comm_ref[0, 1] = wo_ref[:]

        def ring_rdma(t, part, ssem, rsem):
            return pltpu.make_async_remote_copy(
                src_ref=comm_ref.at[t, part],
                dst_ref=comm_ref.at[t + 1, part],
                send_sem=ssem.at[t],
                recv_sem=rsem.at[t],
                device_id=(right,),
                device_id_type=pl.DeviceIdType.MESH,
            )

        wq_rdmas = [ring_rdma(t, 0, sqs, rqs) for t in range(N_DEV - 1)]
        wo_rdmas = [ring_rdma(t, 1, sos, ros) for t in range(N_DEV - 1)]

        def kv_copies(chunk, hl, buf):
            g = chunk * HQ + hl
            cps = []
            for r in range(G):
                cps.append(pltpu.make_async_copy(
                    kr_ref.at[:, r, :, g, :], kraw_ref.at[buf, r],
                    ksem.at[buf]))
                cps.append(pltpu.make_async_copy(
                    vr_ref.at[:, r, :, g, :], vraw_ref.at[buf, r],
                    vsem.at[buf]))
            return cps

        pending = kv_copies(i, 0, 0)
        for c in pending:
            c.start()

        for t in range(N_DEV):
            if t > 0:
                wq_rdmas[t - 1].wait_recv()
            if t < N_DEV - 1:
                wq_rdmas[t].start()
            if t == 0:
                wo_rdmas[0].start()

            j = (i - t) % N_DEV
            wq_c = comm_ref[t, 0]

            qf = jnp.dot(xg_ref[:], wq_c,
                         preferred_element_type=jnp.float32)
            qb_ref[:] = (qf * SCALE).astype(jnp.bfloat16)

            for hl in range(HQ):
                buf = hl % 2
                if hl == HQ // 2 and t > 0:
                    wo_rdmas[t - 1].wait_recv()
                    if t < N_DEV - 1:
                        wo_rdmas[t].start()
                for c in pending:
                    c.wait()
                khg_ref[:] = kraw_ref[buf].reshape(G, GR, DH).astype(
                    jnp.bfloat16)
                vhg_ref[:] = vraw_ref[buf].reshape(G, GR, DH).astype(
                    jnp.bfloat16)
                if hl < HQ - 1:
                    pending = kv_copies(j, hl + 1, 1 - buf)
                elif t < N_DEV - 1:
                    pending = kv_copies((i - t - 1) % N_DEV, 0, 1 - buf)
                else:
                    pending = []
                for c in pending:
                    c.start()
                q = qb_ref[:, hl * DH:(hl + 1) * DH].reshape(G, GR, DH)
                s = lax.dot_general(
                    q, khg_ref[:],
                    (((2,), (2,)), ((0,), (0,))),
                    preferred_element_type=jnp.float32,
                )
                e = jnp.exp(s)
                den = jnp.sum(e, axis=2, keepdims=True)
                c_un = lax.dot_general(
                    e.astype(jnp.bfloat16), vhg_ref[:],
                    (((2,), (1,)), ((0,), (0,))),
                    preferred_element_type=jnp.float32,
                )
                c_n = c_un * (1.0 / den)
                ctx_ref[:, hl * DH:(hl + 1) * DH] = c_n.astype(
                    jnp.bfloat16).reshape(SQ, DH)

            wo_c = comm_ref[t, 1]

            part = jnp.dot(ctx_ref[:], wo_c,
                           preferred_element_type=jnp.float32)
            if t == 0:
                out_ref[:] = part.reshape(G, GR, DM)
            else:
                out_ref[:] = out_ref[:] + part.reshape(G, GR, DM)

        for t in range(N_DEV - 1):
            wq_rdmas[t].wait_send()
            wo_rdmas[t].wait_send()

    out_g = pl.pallas_call(
        body,
        out_shape=jax.ShapeDtypeStruct((G, GR, DM), jnp.float32),
        in_specs=[
            pl.BlockSpec(memory_space=pltpu.MemorySpace.VMEM),
            pl.BlockSpec(memory_space=pltpu.MemorySpace.VMEM),
            pl.BlockSpec(memory_space=pltpu.MemorySpace.VMEM),
            pl.BlockSpec(memory_space=pltpu.MemorySpace.HBM),
            pl.BlockSpec(memory_space=pltpu.MemorySpace.HBM),
        ],
        out_specs=pl.BlockSpec(memory_space=pltpu.MemorySpace.VMEM),
        scratch_shapes=[
            pltpu.VMEM((N_DEV, 2, DM, DC), jnp.bfloat16),
            pltpu.VMEM((SQ, DC), jnp.bfloat16),
            pltpu.VMEM((SQ, DC), jnp.bfloat16),
            pltpu.VMEM((2, G, P, 64, DH), jnp.float32),
            pltpu.VMEM((2, G, P, 64, DH), jnp.float32),
            pltpu.VMEM((G, GR, DH), jnp.bfloat16),
            pltpu.VMEM((G, GR, DH), jnp.bfloat16),
            pltpu.SemaphoreType.DMA((N_DEV - 1,)),
            pltpu.SemaphoreType.DMA((N_DEV - 1,)),
            pltpu.SemaphoreType.DMA((N_DEV - 1,)),
            pltpu.SemaphoreType.DMA((N_DEV - 1,)),
            pltpu.SemaphoreType.DMA((2,)),
            pltpu.SemaphoreType.DMA((2,)),
        ],
        compiler_params=pltpu.CompilerParams(
            collective_id=0,
            vmem_limit_bytes=100 * 1024 * 1024,
        ),
    )(xg, wq_b, wo_b, kr, vr)

    out = out_g.reshape(G, P, 64, DM).swapaxes(0, 1).reshape(1, SQ, DM)
    return out
